# baseline (speedup 1.0000x reference)
import functools

import jax
import jax.numpy as jnp
from jax import lax
from jax.experimental import pallas as pl
from jax.experimental.pallas import tpu as pltpu
from jax.experimental.pallas import tpu_sc as plsc

_IGNORE = -1.0
_B, _L = 16384, 200
_NC, _NS, _LANES = 2, 16, 16
_NW = _NC * _NS
_RW = _B // _NW
_WORDS = _RW * _L
_NVEC = _WORDS // _LANES
_INNER = 16
_NGROUPS = _RW // _LANES


def _sc_body(x_hbm, vals_hbm, len_hbm, buf, lenbuf):
    wid = lax.axis_index("s") * _NC + lax.axis_index("c")
    base = wid * _WORDS

    pltpu.sync_copy(x_hbm.at[pl.ds(base, _WORDS)], buf)

    lane = lax.iota(jnp.int32, _LANES)

    def search_step(g, _):
        rowbase = (g * _LANES + lane) * _L
        lo = jnp.zeros((_LANES,), jnp.int32)
        hi = jnp.full((_LANES,), _L, jnp.int32)
        for _step in range(8):
            active = lo < hi
            mid = (lo + hi) >> 1
            v = plsc.load_gather(buf, [rowbase + mid])
            is_pad = v == _IGNORE
            hi = jnp.where(active & is_pad, mid, hi)
            lo = jnp.where(active & (~is_pad), mid + 1, lo)
        lenbuf[pl.ds(g * _LANES, _LANES)] = lo
        return 0

    lax.fori_loop(0, _NGROUPS, search_step, 0, unroll=False)

    def flat_step(i, _):
        off = i * (_INNER * _LANES)
        for j in range(_INNER):
            x = buf[pl.ds(off + j * _LANES, _LANES)]
            buf[pl.ds(off + j * _LANES, _LANES)] = jnp.where(
                x != _IGNORE, x, 0.0
            )
        return 0

    lax.fori_loop(0, _NVEC // _INNER, flat_step, 0, unroll=False)

    pltpu.sync_copy(buf, vals_hbm.at[pl.ds(base, _WORDS)])
    pltpu.sync_copy(lenbuf, len_hbm.at[pl.ds(wid * _RW, _RW)])


@functools.partial(jax.jit, static_argnames=())
def _dense_to_ragged(x_flat):
    run = pl.kernel(
        _sc_body,
        out_type=(
            jax.ShapeDtypeStruct((_B * _L,), jnp.float32),
            jax.ShapeDtypeStruct((_B,), jnp.int32),
        ),
        mesh=plsc.VectorSubcoreMesh(
            core_axis_name="c", subcore_axis_name="s",
            num_cores=_NC, num_subcores=_NS,
        ),
        scratch_types=(
            pltpu.VMEM((_WORDS,), jnp.float32),
            pltpu.VMEM((_RW,), jnp.int32),
        ),
        compiler_params=pltpu.CompilerParams(needs_layout_passes=False),
        name="dense_to_ragged_sc",
    )
    return run(x_flat)


def kernel(inputs):
    vals_flat, row_lengths = _dense_to_ragged(inputs.reshape(-1))
    return vals_flat.reshape(_B, _L), row_lengths


# speedup vs baseline: 1.0023x; 1.0023x over previous
import functools

import jax
import jax.numpy as jnp
from jax import lax
from jax.experimental import pallas as pl
from jax.experimental.pallas import tpu as pltpu
from jax.experimental.pallas import tpu_sc as plsc

_IGNORE = -1.0
_B, _L = 16384, 200
_NC, _NS, _LANES = 2, 16, 16
_NW = _NC * _NS
_RW = _B // _NW
_WORDS = _RW * _L
_C = 64
_NCH = _RW // _C
_CW = _C * _L
_CVEC = _CW // _LANES
_NBUF = 3


def _sc_body(x_hbm, vals_hbm, len_hbm, buf0, buf1, buf2, lenbuf,
             isem0, isem1, isem2, osem0, osem1, osem2):
    wid = lax.axis_index("s") * _NC + lax.axis_index("c")
    base = wid * _WORDS
    bufs = (buf0, buf1, buf2)
    isems = (isem0, isem1, isem2)
    osems = (osem0, osem1, osem2)
    lane = lax.iota(jnp.int32, _LANES)

    def in_copy(c):
        b = c % _NBUF
        return pltpu.async_copy(
            x_hbm.at[pl.ds(base + c * _CW, _CW)], bufs[b], isems[b])

    def out_copy(c):
        b = c % _NBUF
        return pltpu.async_copy(
            bufs[b], vals_hbm.at[pl.ds(base + c * _CW, _CW)], osems[b])

    def compute(c):
        buf = bufs[c % _NBUF]

        for g in range(_C // _LANES):
            rowbase = (g * _LANES + lane) * _L
            lo = jnp.zeros((_LANES,), jnp.int32)
            hi = jnp.full((_LANES,), _L, jnp.int32)
            for _step in range(8):
                active = lo < hi
                mid = (lo + hi) >> 1
                v = plsc.load_gather(buf, [rowbase + mid])
                is_pad = v == _IGNORE
                hi = jnp.where(active & is_pad, mid, hi)
                lo = jnp.where(active & (~is_pad), mid + 1, lo)
            lenbuf[pl.ds(c * _C + g * _LANES, _LANES)] = lo

        @plsc.parallel_loop(0, _CW, step=_LANES, unroll=8)
        def _flat(i):
            x = buf[pl.ds(i, _LANES)]
            buf[pl.ds(i, _LANES)] = jnp.where(x != _IGNORE, x, 0.0)

    in_h = [None] * _NCH
    out_h = [None] * _NCH
    out_waited = [False] * _NCH
    for c in range(_NBUF):
        in_h[c] = in_copy(c)
    for g in range(_NCH):
        nxt = g + _NBUF - 1
        if g >= 1 and nxt < _NCH:
            out_h[g - 1].wait()
            out_waited[g - 1] = True
            in_h[nxt] = in_copy(nxt)
        in_h[g].wait()
        compute(g)
        out_h[g] = out_copy(g)
    for g in range(_NCH):
        if not out_waited[g]:
            out_h[g].wait()

    pltpu.sync_copy(lenbuf, len_hbm.at[pl.ds(wid * _RW, _RW)])


@functools.partial(jax.jit, static_argnames=())
def _dense_to_ragged(x_flat):
    run = pl.kernel(
        _sc_body,
        out_type=(
            jax.ShapeDtypeStruct((_B * _L,), jnp.float32),
            jax.ShapeDtypeStruct((_B,), jnp.int32),
        ),
        mesh=plsc.VectorSubcoreMesh(
            core_axis_name="c", subcore_axis_name="s",
            num_cores=_NC, num_subcores=_NS,
        ),
        scratch_types=(
            pltpu.VMEM((_CW,), jnp.float32),
            pltpu.VMEM((_CW,), jnp.float32),
            pltpu.VMEM((_CW,), jnp.float32),
            pltpu.VMEM((_RW,), jnp.int32),
            pltpu.SemaphoreType.DMA,
            pltpu.SemaphoreType.DMA,
            pltpu.SemaphoreType.DMA,
            pltpu.SemaphoreType.DMA,
            pltpu.SemaphoreType.DMA,
            pltpu.SemaphoreType.DMA,
        ),
        compiler_params=pltpu.CompilerParams(needs_layout_passes=False),
        name="dense_to_ragged_sc",
    )
    return run(x_flat)


def kernel(inputs):
    vals_flat, row_lengths = _dense_to_ragged(inputs.reshape(-1))
    return vals_flat.reshape(_B, _L), row_lengths
